# Initial kernel scaffold; baseline (speedup 1.0000x reference)
import jax
import jax.numpy as jnp
from jax.experimental import pallas as pl


def kernel(x, edge_index, edge_weight, batch, enc_w, bias_w, dec_w):
    raise NotImplementedError("write your pallas kernel here")


# speedup vs baseline: 2.4994x; 2.4994x over previous
import functools

import jax
import jax.numpy as jnp
from jax import lax
from jax.experimental import pallas as pl
from jax.experimental.pallas import tpu as pltpu
from jax.experimental.pallas import tpu_sc as plsc

N = 10000
E = 320000
D = 128
BETA = 0.9
GAMMA = 0.5
PHANTOM_GRAD = 5

NTILE = 16
ROWS_PER_TILE = 640
N_PAD = NTILE * ROWS_PER_TILE
CHUNK = 128
GROUPS = 20
CH_PER_G = 8
E_PER_TILE = GROUPS * CH_PER_G * CHUNK
E_PAD = NTILE * E_PER_TILE
UROWS = 32
UCHUNKS = ROWS_PER_TILE // UROWS


def _zero_fill(ref):
    def body(i, _):
        for q in range(8):
            ref[i, pl.ds(q * 16, 16)] = jnp.zeros((16,), jnp.float32)
        return 0
    lax.fori_loop(0, ref.shape[0], body, 0)


def _sc_body(src_h, dst_h, w_h, b_h, u_h,
             agg_sh, src_sb, dst_sb, w_sb, rows_v, ub, bb, ab, zb, sem):
    t = lax.axis_index("s")
    row0 = t * ROWS_PER_TILE

    _zero_fill(zb)

    def init_chunk(c, _):
        r = row0 + c * UROWS
        pltpu.sync_copy(b_h.at[pl.ds(r, UROWS)], bb)

        def init_row(i, _):
            for q in range(8):
                s = pl.ds(q * 16, 16)
                ub[i, s] = GAMMA * jnp.maximum(bb[i, s], 0.0)
            return 0
        lax.fori_loop(0, UROWS, init_row, 0)
        pltpu.sync_copy(ub, u_h.at[pl.ds(r, UROWS)])
        pltpu.sync_copy(zb, agg_sh.at[pl.ds(r, UROWS)])
        return 0
    lax.fori_loop(0, UCHUNKS, init_chunk, 0)
    plsc.subcore_barrier()

    def one_round(it, _):
        def group_body(g, _):
            gbase = g * CH_PER_G
            pltpu.sync_copy(src_h.at[t, pl.ds(gbase, CH_PER_G)], src_sb)
            pltpu.sync_copy(dst_h.at[t, pl.ds(gbase, CH_PER_G)], dst_sb)
            pltpu.sync_copy(w_h.at[t, pl.ds(gbase, CH_PER_G)], w_sb)

            def chunk_body(j, _):
                pltpu.async_copy(u_h.at[src_sb.at[j]], rows_v, sem).wait()

                def mul_row16(k16, _):
                    wv = w_sb[j, pl.ds(k16 * 16, 16)]
                    for l in range(16):
                        wk = wv[l]
                        for q in range(8):
                            s = pl.ds(q * 16, 16)
                            rows_v[k16 * 16 + l, s] = rows_v[k16 * 16 + l, s] * wk
                    return 0
                lax.fori_loop(0, CHUNK // 16, mul_row16, 0)
                pltpu.sync_copy(rows_v, agg_sh.at[dst_sb.at[j]], add=True)
                return 0
            lax.fori_loop(0, CH_PER_G, chunk_body, 0)
            return 0
        lax.fori_loop(0, GROUPS, group_body, 0)
        plsc.subcore_barrier()

        def upd_chunk(c, _):
            r = row0 + c * UROWS
            pltpu.sync_copy(u_h.at[pl.ds(r, UROWS)], ub)
            pltpu.sync_copy(b_h.at[pl.ds(r, UROWS)], bb)
            pltpu.sync_copy(agg_sh.at[pl.ds(r, UROWS)], ab)

            def upd_row(i, _):
                for q in range(8):
                    s = pl.ds(q * 16, 16)
                    z = jnp.maximum(BETA * ab[i, s] + bb[i, s], 0.0)
                    ub[i, s] = (1.0 - GAMMA) * ub[i, s] + GAMMA * z
                return 0
            lax.fori_loop(0, UROWS, upd_row, 0)
            pltpu.sync_copy(ub, u_h.at[pl.ds(r, UROWS)])
            pltpu.sync_copy(zb, agg_sh.at[pl.ds(r, UROWS)])
            return 0
        lax.fori_loop(0, UCHUNKS, upd_chunk, 0)
        plsc.subcore_barrier()
        return 0
    lax.fori_loop(0, PHANTOM_GRAD - 1, one_round, 0)


_sc_iterate = functools.partial(
    pl.kernel,
    out_type=jax.ShapeDtypeStruct((N_PAD, D), jnp.float32),
    mesh=plsc.VectorSubcoreMesh(
        core_axis_name="c", subcore_axis_name="s", num_cores=1),
    scratch_types=[
        pltpu.VMEM_SHARED((N_PAD, D), jnp.float32),
        pltpu.VMEM((CH_PER_G, CHUNK), jnp.int32),
        pltpu.VMEM((CH_PER_G, CHUNK), jnp.int32),
        pltpu.VMEM((CH_PER_G, CHUNK), jnp.float32),
        pltpu.VMEM((CHUNK, D), jnp.float32),
        pltpu.VMEM((UROWS, D), jnp.float32),
        pltpu.VMEM((UROWS, D), jnp.float32),
        pltpu.VMEM((UROWS, D), jnp.float32),
        pltpu.VMEM((UROWS, D), jnp.float32),
        pltpu.SemaphoreType.DMA,
    ],
)(_sc_body)


def _mm2_block(x_ref, a_ref, c_ref, o_ref):
    h = jnp.dot(x_ref[...], a_ref[...], preferred_element_type=jnp.float32)
    o_ref[...] = jnp.dot(h, c_ref[...], preferred_element_type=jnp.float32)


_mm2 = pl.pallas_call(
    _mm2_block,
    grid=(10,),
    in_specs=[
        pl.BlockSpec((1000, D), lambda i: (i, 0)),
        pl.BlockSpec((D, D), lambda i: (0, 0)),
        pl.BlockSpec((D, D), lambda i: (0, 0)),
    ],
    out_specs=pl.BlockSpec((1000, D), lambda i: (i, 0)),
    out_shape=jax.ShapeDtypeStruct((N, D), jnp.float32),
)


def _dec_block(u_ref, w_ref, o_ref):
    r = jnp.maximum(u_ref[...], 0.0)
    o_ref[...] = jnp.dot(r, w_ref[...], preferred_element_type=jnp.float32)


_decode = pl.pallas_call(
    _dec_block,
    grid=(10,),
    in_specs=[
        pl.BlockSpec((1000, D), lambda i: (i, 0)),
        pl.BlockSpec((D, D), lambda i: (0, 0)),
    ],
    out_specs=pl.BlockSpec((1000, D), lambda i: (i, 0)),
    out_shape=jax.ShapeDtypeStruct((N, D), jnp.float32),
)


def kernel(x, edge_index, edge_weight, batch, enc_w, bias_w, dec_w):
    b = _mm2(x, enc_w.T, bias_w.T)
    b_pad = jnp.pad(b, ((0, N_PAD - N), (0, 0)))

    pad = E_PAD - E
    src = jnp.pad(edge_index[0], (0, pad)).reshape(NTILE, GROUPS * CH_PER_G, CHUNK)
    dst = jnp.pad(edge_index[1], (0, pad)).reshape(NTILE, GROUPS * CH_PER_G, CHUNK)
    w = jnp.pad(edge_weight, (0, pad)).reshape(NTILE, GROUPS * CH_PER_G, CHUNK)

    u_pad = _sc_iterate(src, dst, w, b_pad)
    u = u_pad[:N]
    out = _decode(u, dec_w.T)
    return (out, u)


# speedup vs baseline: 2.5059x; 1.0026x over previous
import functools

import jax
import jax.numpy as jnp
from jax import lax
from jax.experimental import pallas as pl
from jax.experimental.pallas import tpu as pltpu
from jax.experimental.pallas import tpu_sc as plsc

N = 10000
E = 320000
D = 128
BETA = 0.9
GAMMA = 0.5
PHANTOM_GRAD = 5

NTILE = 16
ROWS_PER_TILE = 640
N_PAD = NTILE * ROWS_PER_TILE
CHUNK = 128
GROUPS = 20
CH_PER_G = 8
E_PER_TILE = GROUPS * CH_PER_G * CHUNK
E_PAD = NTILE * E_PER_TILE
UROWS = 32
UCHUNKS = ROWS_PER_TILE // UROWS


def _zero_fill(ref):
    def body(i, _):
        for q in range(8):
            ref[i, pl.ds(q * 16, 16)] = jnp.zeros((16,), jnp.float32)
        return 0
    lax.fori_loop(0, ref.shape[0], body, 0)


def _sc_body(src_h, dst_h, w_h, b_h, u_h,
             agg_sh, src_sb, dst_sb, w_sb, rows_v, ub, bb, ab, zb, sem):
    t = lax.axis_index("s")
    row0 = t * ROWS_PER_TILE

    _zero_fill(zb)

    def init_chunk(c, _):
        r = row0 + c * UROWS
        pltpu.sync_copy(b_h.at[pl.ds(r, UROWS)], bb)

        @plsc.parallel_loop(0, UROWS, unroll=2)
        def init_row(i):
            for q in range(8):
                s = pl.ds(q * 16, 16)
                ub[i, s] = GAMMA * jnp.maximum(bb[i, s], 0.0)
        pltpu.sync_copy(ub, u_h.at[pl.ds(r, UROWS)])
        pltpu.sync_copy(zb, agg_sh.at[pl.ds(r, UROWS)])
        return 0
    lax.fori_loop(0, UCHUNKS, init_chunk, 0)
    plsc.subcore_barrier()

    def one_round(it, _):
        def group_body(g, _):
            gbase = g * CH_PER_G
            pltpu.sync_copy(src_h.at[t, pl.ds(gbase, CH_PER_G)], src_sb)
            pltpu.sync_copy(dst_h.at[t, pl.ds(gbase, CH_PER_G)], dst_sb)
            pltpu.sync_copy(w_h.at[t, pl.ds(gbase, CH_PER_G)], w_sb)

            def chunk_body(j, _):
                pltpu.async_copy(u_h.at[src_sb.at[j]], rows_v, sem).wait()

                @plsc.parallel_loop(0, CHUNK // 16, unroll=2)
                def mul_row16(k16):
                    wv = w_sb[j, pl.ds(k16 * 16, 16)]
                    for l in range(16):
                        wk = wv[l]
                        for q in range(8):
                            s = pl.ds(q * 16, 16)
                            rows_v[k16 * 16 + l, s] = rows_v[k16 * 16 + l, s] * wk
                pltpu.sync_copy(rows_v, agg_sh.at[dst_sb.at[j]], add=True)
                return 0
            lax.fori_loop(0, CH_PER_G, chunk_body, 0)
            return 0
        lax.fori_loop(0, GROUPS, group_body, 0)
        plsc.subcore_barrier()

        def upd_chunk(c, _):
            r = row0 + c * UROWS
            pltpu.sync_copy(u_h.at[pl.ds(r, UROWS)], ub)
            pltpu.sync_copy(b_h.at[pl.ds(r, UROWS)], bb)
            pltpu.sync_copy(agg_sh.at[pl.ds(r, UROWS)], ab)

            @plsc.parallel_loop(0, UROWS, unroll=2)
            def upd_row(i):
                for q in range(8):
                    s = pl.ds(q * 16, 16)
                    z = jnp.maximum(BETA * ab[i, s] + bb[i, s], 0.0)
                    ub[i, s] = (1.0 - GAMMA) * ub[i, s] + GAMMA * z
            pltpu.sync_copy(ub, u_h.at[pl.ds(r, UROWS)])
            pltpu.sync_copy(zb, agg_sh.at[pl.ds(r, UROWS)])
            return 0
        lax.fori_loop(0, UCHUNKS, upd_chunk, 0)
        plsc.subcore_barrier()
        return 0
    lax.fori_loop(0, PHANTOM_GRAD - 1, one_round, 0)


_sc_iterate = functools.partial(
    pl.kernel,
    out_type=jax.ShapeDtypeStruct((N_PAD, D), jnp.float32),
    mesh=plsc.VectorSubcoreMesh(
        core_axis_name="c", subcore_axis_name="s", num_cores=1),
    scratch_types=[
        pltpu.VMEM_SHARED((N_PAD, D), jnp.float32),
        pltpu.VMEM((CH_PER_G, CHUNK), jnp.int32),
        pltpu.VMEM((CH_PER_G, CHUNK), jnp.int32),
        pltpu.VMEM((CH_PER_G, CHUNK), jnp.float32),
        pltpu.VMEM((CHUNK, D), jnp.float32),
        pltpu.VMEM((UROWS, D), jnp.float32),
        pltpu.VMEM((UROWS, D), jnp.float32),
        pltpu.VMEM((UROWS, D), jnp.float32),
        pltpu.VMEM((UROWS, D), jnp.float32),
        pltpu.SemaphoreType.DMA,
    ],
)(_sc_body)


def _mm2_block(x_ref, a_ref, c_ref, o_ref):
    h = jnp.dot(x_ref[...], a_ref[...], preferred_element_type=jnp.float32)
    o_ref[...] = jnp.dot(h, c_ref[...], preferred_element_type=jnp.float32)


_mm2 = pl.pallas_call(
    _mm2_block,
    grid=(10,),
    in_specs=[
        pl.BlockSpec((1000, D), lambda i: (i, 0)),
        pl.BlockSpec((D, D), lambda i: (0, 0)),
        pl.BlockSpec((D, D), lambda i: (0, 0)),
    ],
    out_specs=pl.BlockSpec((1000, D), lambda i: (i, 0)),
    out_shape=jax.ShapeDtypeStruct((N, D), jnp.float32),
)


def _dec_block(u_ref, w_ref, o_ref):
    r = jnp.maximum(u_ref[...], 0.0)
    o_ref[...] = jnp.dot(r, w_ref[...], preferred_element_type=jnp.float32)


_decode = pl.pallas_call(
    _dec_block,
    grid=(10,),
    in_specs=[
        pl.BlockSpec((1000, D), lambda i: (i, 0)),
        pl.BlockSpec((D, D), lambda i: (0, 0)),
    ],
    out_specs=pl.BlockSpec((1000, D), lambda i: (i, 0)),
    out_shape=jax.ShapeDtypeStruct((N, D), jnp.float32),
)


def kernel(x, edge_index, edge_weight, batch, enc_w, bias_w, dec_w):
    b = _mm2(x, enc_w.T, bias_w.T)
    b_pad = jnp.pad(b, ((0, N_PAD - N), (0, 0)))

    pad = E_PAD - E
    src = jnp.pad(edge_index[0], (0, pad)).reshape(NTILE, GROUPS * CH_PER_G, CHUNK)
    dst = jnp.pad(edge_index[1], (0, pad)).reshape(NTILE, GROUPS * CH_PER_G, CHUNK)
    w = jnp.pad(edge_weight, (0, pad)).reshape(NTILE, GROUPS * CH_PER_G, CHUNK)

    u_pad = _sc_iterate(src, dst, w, b_pad)
    u = u_pad[:N]
    out = _decode(u, dec_w.T)
    return (out, u)


# speedup vs baseline: 2.5408x; 1.0139x over previous
import functools

import jax
import jax.numpy as jnp
from jax import lax
from jax.experimental import pallas as pl
from jax.experimental.pallas import tpu as pltpu
from jax.experimental.pallas import tpu_sc as plsc

N = 10000
E = 320000
D = 128
BETA = 0.9
GAMMA = 0.5
PHANTOM_GRAD = 5

NTILE = 16
ROWS_PER_TILE = 640
N_PAD = NTILE * ROWS_PER_TILE
CHUNK = 128
GROUPS = 20
CH_PER_G = 8
E_PER_TILE = GROUPS * CH_PER_G * CHUNK
E_PAD = NTILE * E_PER_TILE
UROWS = 16
UCHUNKS = ROWS_PER_TILE // UROWS


def _zero_fill(ref):
    def body(i, _):
        for q in range(8):
            ref[i, pl.ds(q * 16, 16)] = jnp.zeros((16,), jnp.float32)
        return 0
    lax.fori_loop(0, ref.shape[0], body, 0)


def _sc_body(src_h, dst_h, w_h, b_h, u_h,
             agg_sh, src_sb, dst_sb, w_sb, rows_v, rows_w, ub, bb, ab, zb, sem, sem2):
    t = lax.axis_index("s")
    row0 = t * ROWS_PER_TILE

    _zero_fill(zb)

    def init_chunk(c, _):
        r = row0 + c * UROWS
        pltpu.sync_copy(b_h.at[pl.ds(r, UROWS)], bb)

        @plsc.parallel_loop(0, UROWS, unroll=2)
        def init_row(i):
            for q in range(8):
                s = pl.ds(q * 16, 16)
                ub[i, s] = GAMMA * jnp.maximum(bb[i, s], 0.0)
        pltpu.sync_copy(ub, u_h.at[pl.ds(r, UROWS)])
        pltpu.sync_copy(zb, agg_sh.at[pl.ds(r, UROWS)])
        return 0
    lax.fori_loop(0, UCHUNKS, init_chunk, 0)
    plsc.subcore_barrier()

    def one_round(it, _):
        def group_body(g, _):
            gbase = g * CH_PER_G
            pltpu.sync_copy(src_h.at[t, pl.ds(gbase, CH_PER_G)], src_sb)
            pltpu.sync_copy(dst_h.at[t, pl.ds(gbase, CH_PER_G)], dst_sb)
            pltpu.sync_copy(w_h.at[t, pl.ds(gbase, CH_PER_G)], w_sb)

            def pair_body(m, _):
                a = 2 * m
                b = a + 1
                da = pltpu.async_copy(u_h.at[src_sb.at[a]], rows_v, sem)
                db = pltpu.async_copy(u_h.at[src_sb.at[b]], rows_w, sem2)
                da.wait()

                @plsc.parallel_loop(0, CHUNK // 16, unroll=2)
                def mul_a(k16):
                    wv = w_sb[a, pl.ds(k16 * 16, 16)]
                    for l in range(16):
                        wk = wv[l]
                        for q in range(8):
                            s = pl.ds(q * 16, 16)
                            rows_v[k16 * 16 + l, s] = rows_v[k16 * 16 + l, s] * wk
                db.wait()

                @plsc.parallel_loop(0, CHUNK // 16, unroll=2)
                def mul_b(k16):
                    wv = w_sb[b, pl.ds(k16 * 16, 16)]
                    for l in range(16):
                        wk = wv[l]
                        for q in range(8):
                            s = pl.ds(q * 16, 16)
                            rows_w[k16 * 16 + l, s] = rows_w[k16 * 16 + l, s] * wk
                pltpu.sync_copy(rows_v, agg_sh.at[dst_sb.at[a]], add=True)
                pltpu.sync_copy(rows_w, agg_sh.at[dst_sb.at[b]], add=True)
                return 0
            lax.fori_loop(0, CH_PER_G // 2, pair_body, 0)
            return 0
        lax.fori_loop(0, GROUPS, group_body, 0)
        plsc.subcore_barrier()

        def upd_chunk(c, _):
            r = row0 + c * UROWS
            pltpu.sync_copy(u_h.at[pl.ds(r, UROWS)], ub)
            pltpu.sync_copy(b_h.at[pl.ds(r, UROWS)], bb)
            pltpu.sync_copy(agg_sh.at[pl.ds(r, UROWS)], ab)

            @plsc.parallel_loop(0, UROWS, unroll=2)
            def upd_row(i):
                for q in range(8):
                    s = pl.ds(q * 16, 16)
                    z = jnp.maximum(BETA * ab[i, s] + bb[i, s], 0.0)
                    ub[i, s] = (1.0 - GAMMA) * ub[i, s] + GAMMA * z
            pltpu.sync_copy(ub, u_h.at[pl.ds(r, UROWS)])
            pltpu.sync_copy(zb, agg_sh.at[pl.ds(r, UROWS)])
            return 0
        lax.fori_loop(0, UCHUNKS, upd_chunk, 0)
        plsc.subcore_barrier()
        return 0
    lax.fori_loop(0, PHANTOM_GRAD - 1, one_round, 0)


_sc_iterate = functools.partial(
    pl.kernel,
    out_type=jax.ShapeDtypeStruct((N_PAD, D), jnp.float32),
    mesh=plsc.VectorSubcoreMesh(
        core_axis_name="c", subcore_axis_name="s", num_cores=1),
    scratch_types=[
        pltpu.VMEM_SHARED((N_PAD, D), jnp.float32),
        pltpu.VMEM((CH_PER_G, CHUNK), jnp.int32),
        pltpu.VMEM((CH_PER_G, CHUNK), jnp.int32),
        pltpu.VMEM((CH_PER_G, CHUNK), jnp.float32),
        pltpu.VMEM((CHUNK, D), jnp.float32),
        pltpu.VMEM((CHUNK, D), jnp.float32),
        pltpu.VMEM((UROWS, D), jnp.float32),
        pltpu.VMEM((UROWS, D), jnp.float32),
        pltpu.VMEM((UROWS, D), jnp.float32),
        pltpu.VMEM((UROWS, D), jnp.float32),
        pltpu.SemaphoreType.DMA,
        pltpu.SemaphoreType.DMA,
    ],
)(_sc_body)


def _mm2_block(x_ref, a_ref, c_ref, o_ref):
    h = jnp.dot(x_ref[...], a_ref[...], preferred_element_type=jnp.float32)
    o_ref[...] = jnp.dot(h, c_ref[...], preferred_element_type=jnp.float32)


_mm2 = pl.pallas_call(
    _mm2_block,
    grid=(10,),
    in_specs=[
        pl.BlockSpec((1000, D), lambda i: (i, 0)),
        pl.BlockSpec((D, D), lambda i: (0, 0)),
        pl.BlockSpec((D, D), lambda i: (0, 0)),
    ],
    out_specs=pl.BlockSpec((1000, D), lambda i: (i, 0)),
    out_shape=jax.ShapeDtypeStruct((N, D), jnp.float32),
)


def _dec_block(u_ref, w_ref, o_ref):
    r = jnp.maximum(u_ref[...], 0.0)
    o_ref[...] = jnp.dot(r, w_ref[...], preferred_element_type=jnp.float32)


_decode = pl.pallas_call(
    _dec_block,
    grid=(10,),
    in_specs=[
        pl.BlockSpec((1000, D), lambda i: (i, 0)),
        pl.BlockSpec((D, D), lambda i: (0, 0)),
    ],
    out_specs=pl.BlockSpec((1000, D), lambda i: (i, 0)),
    out_shape=jax.ShapeDtypeStruct((N, D), jnp.float32),
)


def kernel(x, edge_index, edge_weight, batch, enc_w, bias_w, dec_w):
    b = _mm2(x, enc_w.T, bias_w.T)
    b_pad = jnp.pad(b, ((0, N_PAD - N), (0, 0)))

    pad = E_PAD - E
    src = jnp.pad(edge_index[0], (0, pad)).reshape(NTILE, GROUPS * CH_PER_G, CHUNK)
    dst = jnp.pad(edge_index[1], (0, pad)).reshape(NTILE, GROUPS * CH_PER_G, CHUNK)
    w = jnp.pad(edge_weight, (0, pad)).reshape(NTILE, GROUPS * CH_PER_G, CHUNK)

    u_pad = _sc_iterate(src, dst, w, b_pad)
    u = u_pad[:N]
    out = _decode(u, dec_w.T)
    return (out, u)


# speedup vs baseline: 3.0869x; 1.2149x over previous
import functools

import jax
import jax.numpy as jnp
from jax import lax
from jax.experimental import pallas as pl
from jax.experimental.pallas import tpu as pltpu
from jax.experimental.pallas import tpu_sc as plsc

N = 10000
E = 320000
D = 128
BETA = 0.9
GAMMA = 0.5
PHANTOM_GRAD = 5

NCORE = 2
NTILE = 16
NW = NCORE * NTILE
ROWS_PER_TILE = 640
N_PAD = NTILE * ROWS_PER_TILE
CHUNK = 128
GROUPS = 10
CH_PER_G = 8
E_PER_W = GROUPS * CH_PER_G * CHUNK
E_PAD = NW * E_PER_W
UROWS = 16
UPD_ROWS_PER_W = N_PAD // NW
UPD_CHUNKS = UPD_ROWS_PER_W // UROWS

_MESH = plsc.VectorSubcoreMesh(
    core_axis_name="c", subcore_axis_name="s", num_cores=NCORE)


def _scatter_body(src_h, dst_h, w_h, u_h, z_h, part_h,
                  agg_sh, src_sb, dst_sb, w_sb, rows_v, rows_w, sem, sem2):
    c = lax.axis_index("c")
    t = lax.axis_index("s")
    w32 = c * NTILE + t
    row0 = t * ROWS_PER_TILE

    pltpu.sync_copy(z_h, agg_sh.at[pl.ds(row0, ROWS_PER_TILE)])
    plsc.subcore_barrier()

    def group_body(g, _):
        pltpu.sync_copy(src_h.at[w32, g], src_sb)
        pltpu.sync_copy(dst_h.at[w32, g], dst_sb)
        pltpu.sync_copy(w_h.at[w32, g], w_sb)

        def pair_body(m, _):
            a = 2 * m
            b = a + 1
            da = pltpu.async_copy(u_h.at[src_sb.at[a]], rows_v, sem)
            db = pltpu.async_copy(u_h.at[src_sb.at[b]], rows_w, sem2)
            da.wait()

            @plsc.parallel_loop(0, CHUNK // 16, unroll=2)
            def mul_a(k16):
                wv = w_sb[a, pl.ds(k16 * 16, 16)]
                for l in range(16):
                    wk = wv[l]
                    for q in range(8):
                        s = pl.ds(q * 16, 16)
                        rows_v[k16 * 16 + l, s] = rows_v[k16 * 16 + l, s] * wk
            db.wait()

            @plsc.parallel_loop(0, CHUNK // 16, unroll=2)
            def mul_b(k16):
                wv = w_sb[b, pl.ds(k16 * 16, 16)]
                for l in range(16):
                    wk = wv[l]
                    for q in range(8):
                        s = pl.ds(q * 16, 16)
                        rows_w[k16 * 16 + l, s] = rows_w[k16 * 16 + l, s] * wk
            pltpu.sync_copy(rows_v, agg_sh.at[dst_sb.at[a]], add=True)
            pltpu.sync_copy(rows_w, agg_sh.at[dst_sb.at[b]], add=True)
            return 0
        lax.fori_loop(0, CH_PER_G // 2, pair_body, 0)
        return 0
    lax.fori_loop(0, GROUPS, group_body, 0)
    plsc.subcore_barrier()

    pltpu.sync_copy(agg_sh.at[pl.ds(row0, ROWS_PER_TILE)],
                    part_h.at[c, pl.ds(row0, ROWS_PER_TILE)])


_sc_scatter = functools.partial(
    pl.kernel,
    out_type=jax.ShapeDtypeStruct((NCORE, N_PAD, D), jnp.float32),
    mesh=_MESH,
    scratch_types=[
        pltpu.VMEM_SHARED((N_PAD, D), jnp.float32),
        pltpu.VMEM((CH_PER_G, CHUNK), jnp.int32),
        pltpu.VMEM((CH_PER_G, CHUNK), jnp.int32),
        pltpu.VMEM((CH_PER_G, CHUNK), jnp.float32),
        pltpu.VMEM((CHUNK, D), jnp.float32),
        pltpu.VMEM((CHUNK, D), jnp.float32),
        pltpu.SemaphoreType.DMA,
        pltpu.SemaphoreType.DMA,
    ],
)(_scatter_body)


def _update_body(u_h, b_h, part_h, uo_h, ub, bb, p0, p1, sem):
    c = lax.axis_index("c")
    t = lax.axis_index("s")
    w32 = c * NTILE + t
    base = w32 * UPD_ROWS_PER_W

    def upd_chunk(k, _):
        r = base + k * UROWS
        d1 = pltpu.async_copy(u_h.at[pl.ds(r, UROWS)], ub, sem)
        d2 = pltpu.async_copy(b_h.at[pl.ds(r, UROWS)], bb, sem)
        d3 = pltpu.async_copy(part_h.at[0, pl.ds(r, UROWS)], p0, sem)
        d4 = pltpu.async_copy(part_h.at[1, pl.ds(r, UROWS)], p1, sem)
        d1.wait()
        d2.wait()
        d3.wait()
        d4.wait()

        @plsc.parallel_loop(0, UROWS, unroll=2)
        def upd_row(i):
            for q in range(8):
                s = pl.ds(q * 16, 16)
                agg = p0[i, s] + p1[i, s]
                z = jnp.maximum(BETA * agg + bb[i, s], 0.0)
                ub[i, s] = (1.0 - GAMMA) * ub[i, s] + GAMMA * z
        pltpu.sync_copy(ub, uo_h.at[pl.ds(r, UROWS)])
        return 0
    lax.fori_loop(0, UPD_CHUNKS, upd_chunk, 0)


_sc_update = functools.partial(
    pl.kernel,
    out_type=jax.ShapeDtypeStruct((N_PAD, D), jnp.float32),
    mesh=_MESH,
    scratch_types=[
        pltpu.VMEM((UROWS, D), jnp.float32),
        pltpu.VMEM((UROWS, D), jnp.float32),
        pltpu.VMEM((UROWS, D), jnp.float32),
        pltpu.VMEM((UROWS, D), jnp.float32),
        pltpu.SemaphoreType.DMA,
    ],
)(_update_body)


def _enc_block(x_ref, a_ref, c_ref, b_ref, u1_ref):
    h = jnp.dot(x_ref[...], a_ref[...], preferred_element_type=jnp.float32)
    b = jnp.dot(h, c_ref[...], preferred_element_type=jnp.float32)
    b_ref[...] = b
    u1_ref[...] = GAMMA * jnp.maximum(b, 0.0)


_encode = pl.pallas_call(
    _enc_block,
    grid=(10,),
    in_specs=[
        pl.BlockSpec((1000, D), lambda i: (i, 0)),
        pl.BlockSpec((D, D), lambda i: (0, 0)),
        pl.BlockSpec((D, D), lambda i: (0, 0)),
    ],
    out_specs=[
        pl.BlockSpec((1000, D), lambda i: (i, 0)),
        pl.BlockSpec((1000, D), lambda i: (i, 0)),
    ],
    out_shape=[
        jax.ShapeDtypeStruct((N, D), jnp.float32),
        jax.ShapeDtypeStruct((N, D), jnp.float32),
    ],
)


def _dec_block(u_ref, w_ref, o_ref):
    r = jnp.maximum(u_ref[...], 0.0)
    o_ref[...] = jnp.dot(r, w_ref[...], preferred_element_type=jnp.float32)


_decode = pl.pallas_call(
    _dec_block,
    grid=(10,),
    in_specs=[
        pl.BlockSpec((1000, D), lambda i: (i, 0)),
        pl.BlockSpec((D, D), lambda i: (0, 0)),
    ],
    out_specs=pl.BlockSpec((1000, D), lambda i: (i, 0)),
    out_shape=jax.ShapeDtypeStruct((N, D), jnp.float32),
)


def kernel(x, edge_index, edge_weight, batch, enc_w, bias_w, dec_w):
    b, u1 = _encode(x, enc_w.T, bias_w.T)
    b_pad = jnp.pad(b, ((0, N_PAD - N), (0, 0)))
    u = jnp.pad(u1, ((0, N_PAD - N), (0, 0)))

    pad = E_PAD - E
    src = jnp.pad(edge_index[0], (0, pad)).reshape(NW, GROUPS, CH_PER_G, CHUNK)
    dst = jnp.pad(edge_index[1], (0, pad)).reshape(NW, GROUPS, CH_PER_G, CHUNK)
    w = jnp.pad(edge_weight, (0, pad)).reshape(NW, GROUPS, CH_PER_G, CHUNK)
    zeros = jnp.zeros((ROWS_PER_TILE, D), jnp.float32)

    for _ in range(PHANTOM_GRAD - 1):
        parts = _sc_scatter(src, dst, w, u, zeros)
        u = _sc_update(u, b_pad, parts)

    u = u[:N]
    out = _decode(u, dec_w.T)
    return (out, u)
